# baseline (speedup 1.0000x reference)
import jax
import jax.numpy as jnp
from jax.experimental import pallas as pl

_NS = 4
_BB = 16


def _mm_kernel(*refs):
    x_refs = refs[:_NS]
    e_ref = refs[_NS]
    o_refs = refs[_NS + 1:]
    e = e_ref[...].astype(jnp.bfloat16)
    for s in range(_NS):
        x = x_refs[s][...]
        BB, S, V = x.shape
        pad = jnp.zeros((BB, 56 - S, V), dtype=x.dtype)
        x2 = jnp.concatenate([x, pad], axis=1).reshape(BB * 56, V)
        y = jnp.dot(x2.astype(jnp.bfloat16), e,
                    preferred_element_type=jnp.float32)
        o_refs[s][...] = y.reshape(BB, 56, -1)[:, :S, :]


def kernel(inputs, embedding):
    B, S, V = inputs.shape
    D = embedding.shape[1]
    steps = B // (_NS * _BB)

    in_specs = [
        pl.BlockSpec((_BB, S, V), lambda i, s=s: (s * steps + i, 0, 0))
        for s in range(_NS)
    ]
    in_specs.append(pl.BlockSpec((V, D), lambda i: (0, 0)))

    outs = pl.pallas_call(
        _mm_kernel,
        grid=(steps,),
        in_specs=in_specs,
        out_specs=[pl.BlockSpec((_BB, S, D), lambda i: (i, 0, 0))
                   for _ in range(_NS)],
        out_shape=[jax.ShapeDtypeStruct((B // _NS, S, D), jnp.float32)
                   for _ in range(_NS)],
    )(*([inputs] * _NS), embedding)
    return jnp.concatenate(outs, axis=0)


# speedup vs baseline: 5.1013x; 5.1013x over previous
import jax
import jax.numpy as jnp
from jax.experimental import pallas as pl

_SB = 2


def _mm_kernel(x_ref, e_ref, o_ref):
    e = e_ref[...].astype(jnp.bfloat16)
    for s in range(_SB):
        x = x_ref[s].astype(jnp.bfloat16)
        o_ref[s] = jax.lax.dot_general(
            e, x, (((0,), (0,)), ((), ())),
            preferred_element_type=jnp.float32)


def kernel(inputs, embedding):
    B, S, V = inputs.shape
    D = embedding.shape[1]

    xt = jnp.transpose(inputs, (1, 2, 0))

    ot = pl.pallas_call(
        _mm_kernel,
        grid=(S // _SB,),
        in_specs=[
            pl.BlockSpec((_SB, V, B), lambda i: (i, 0, 0)),
            pl.BlockSpec((V, D), lambda i: (0, 0)),
        ],
        out_specs=pl.BlockSpec((_SB, D, B), lambda i: (i, 0, 0)),
        out_shape=jax.ShapeDtypeStruct((S, D, B), jnp.float32),
    )(xt, embedding)
    return jnp.transpose(ot, (2, 0, 1))
